# Initial kernel scaffold; baseline (speedup 1.0000x reference)
import jax
import jax.numpy as jnp
from jax.experimental import pallas as pl


def kernel(vision_features, W, b, num_tokens):
    raise NotImplementedError("write your pallas kernel here")


# speedup vs baseline: 1.1287x; 1.1287x over previous
import functools

import jax
import jax.numpy as jnp
import numpy as np
from jax import lax
from jax.experimental import pallas as pl
from jax.experimental.pallas import tpu as pltpu
from jax.experimental.pallas import tpu_sc as plsc

B = 32
S = 4096
H = 768
K = 256
NC = 2
NS = 16
SB = 2048
NV = S // 16
INT_MIN = np.int32(-2147483648)


def _score_body(w_ref, f_ref, o_ref):
    f = f_ref[0]
    w = w_ref[0]
    o_ref[...] = jnp.dot(f, w, preferred_element_type=jnp.float32)[None, None]


def _scores(features, W):
    return pl.pallas_call(
        _score_body,
        grid=(B, S // SB),
        in_specs=[
            pl.BlockSpec((1, H), lambda b, s: (0, 0)),
            pl.BlockSpec((1, SB, H), lambda b, s: (b, s, 0)),
        ],
        out_specs=pl.BlockSpec((1, 1, SB), lambda b, s: (b, 0, s)),
        out_shape=jax.ShapeDtypeStruct((B, 1, S), jnp.float32),
        compiler_params=pltpu.CompilerParams(
            dimension_semantics=("parallel", "arbitrary"),
        ),
    )(W, features).reshape(B, S)


def _sc_body(feat_hbm, logits_hbm, sel_hbm, idx_hbm, scr_hbm,
             logits_v, ukeys_v, aval_v, aidx_v, bval_v, bidx_v,
             gidx_v, rows_v, outsc_v, outix_v, sem):
    wid = lax.axis_index("s") * NC + lax.axis_index("c")
    iota16 = lax.iota(jnp.int32, 16)

    pltpu.sync_copy(logits_hbm.at[wid], logits_v)

    def p1(i, mvec):
        x = logits_v[pl.ds(i * 16, 16)]
        kb = lax.bitcast_convert_type(x, jnp.int32)
        u = kb ^ ((kb >> 31) | INT_MIN)
        ukeys_v[pl.ds(i * 16, 16)] = lax.bitcast_convert_type(u, jnp.uint32)
        return jnp.maximum(mvec, x)
    mvec = lax.fori_loop(0, NV, p1, jnp.full((16,), -jnp.inf, jnp.float32))
    m = jnp.max(mvec)

    def p2(i, dvec):
        x = logits_v[pl.ds(i * 16, 16)]
        return dvec + jnp.exp(x - m)
    dvec = lax.fori_loop(0, NV, p2, jnp.zeros((16,), jnp.float32))
    denom = jnp.sum(dvec)

    def bs(bi, t):
        cand = t | (jnp.uint32(1) << (jnp.uint32(31) - bi.astype(jnp.uint32)))
        def cnt_body(i, c):
            u = ukeys_v[pl.ds(i * 16, 16)]
            return c + jnp.where(u >= cand, 1, 0).astype(jnp.int32)
        cvec = lax.fori_loop(0, NV, cnt_body, jnp.zeros((16,), jnp.int32))
        cnt = jnp.sum(cvec)
        return jnp.where(cnt >= K, cand, t)
    t = lax.fori_loop(0, 32, bs, jnp.uint32(0))

    def comp_gt(i, off):
        u = ukeys_v[pl.ds(i * 16, 16)]
        mask = u > t
        ones = jnp.where(mask, 1, 0).astype(jnp.int32)
        pos = off + plsc.cumsum(ones) - 1
        plsc.store_scatter(aval_v, [pos], logits_v[pl.ds(i * 16, 16)], mask=mask)
        plsc.store_scatter(aidx_v, [pos], i * 16 + iota16, mask=mask)
        return off + jnp.sum(ones)
    cnt_gt = lax.fori_loop(0, NV, comp_gt, jnp.int32(0))

    def comp_eq(i, off):
        u = ukeys_v[pl.ds(i * 16, 16)]
        mask = u == t
        ones = jnp.where(mask, 1, 0).astype(jnp.int32)
        pos = off + plsc.cumsum(ones) - 1
        wmask = mask & (pos < K)
        plsc.store_scatter(aval_v, [pos], logits_v[pl.ds(i * 16, 16)], mask=wmask)
        plsc.store_scatter(aidx_v, [pos], i * 16 + iota16, mask=wmask)
        return off + jnp.sum(ones)
    lax.fori_loop(0, NV, comp_eq, cnt_gt)

    k = 2
    while k <= K:
        j = k // 2
        while j >= 1:
            def stage(g, _, j=j, k=k):
                ivec = g * 16 + iota16
                pvec = ivec ^ j
                av = aval_v[pl.ds(g * 16, 16)]
                ai = aidx_v[pl.ds(g * 16, 16)]
                bv = plsc.load_gather(aval_v, [pvec])
                bi = plsc.load_gather(aidx_v, [pvec])
                a_first = (av > bv) | ((av == bv) & (ai < bi))
                want_first = ((ivec & k) == 0) == ((ivec & j) == 0)
                cond = want_first == a_first
                bval_v[pl.ds(g * 16, 16)] = jnp.where(cond, av, bv)
                bidx_v[pl.ds(g * 16, 16)] = jnp.where(cond, ai, bi)
                return 0
            lax.fori_loop(0, K // 16, stage, 0)
            def copyb(g, _):
                aval_v[pl.ds(g * 16, 16)] = bval_v[pl.ds(g * 16, 16)]
                aidx_v[pl.ds(g * 16, 16)] = bidx_v[pl.ds(g * 16, 16)]
                return 0
            lax.fori_loop(0, K // 16, copyb, 0)
            j //= 2
        k *= 2

    def outp(g, _):
        v = aval_v[pl.ds(g * 16, 16)]
        outsc_v[pl.ds(g * 16, 16)] = jnp.exp(v - m) / denom
        outix_v[pl.ds(g * 16, 16)] = aidx_v[pl.ds(g * 16, 16)]
        return 0
    lax.fori_loop(0, K // 16, outp, 0)
    pltpu.sync_copy(outsc_v, scr_hbm.at[wid])
    pltpu.sync_copy(outix_v, idx_hbm.at[wid])

    base = wid * S
    for c in range(2):
        for g2 in range(8):
            gidx_v[c, pl.ds(g2 * 16, 16)] = (
                base + aidx_v[pl.ds(c * 128 + g2 * 16, 16)])
        pltpu.async_copy(feat_hbm.at[gidx_v.at[c]], rows_v, sem).wait()
        pltpu.sync_copy(rows_v, sel_hbm.at[wid, pl.ds(c * 128, 128)])


@functools.cache
def _build_select():
    return pl.kernel(
        _sc_body,
        out_type=(
            jax.ShapeDtypeStruct((B, K, H), jnp.float32),
            jax.ShapeDtypeStruct((B, K), jnp.int32),
            jax.ShapeDtypeStruct((B, K), jnp.float32),
        ),
        mesh=plsc.VectorSubcoreMesh(core_axis_name="c", subcore_axis_name="s",
                                    num_cores=NC, num_subcores=NS),
        scratch_types=[
            pltpu.VMEM((S,), jnp.float32),
            pltpu.VMEM((S,), jnp.uint32),
            pltpu.VMEM((K + 16,), jnp.float32),
            pltpu.VMEM((K + 16,), jnp.int32),
            pltpu.VMEM((K,), jnp.float32),
            pltpu.VMEM((K,), jnp.int32),
            pltpu.VMEM((2, 128), jnp.int32),
            pltpu.VMEM((128, H), jnp.float32),
            pltpu.VMEM((K,), jnp.float32),
            pltpu.VMEM((K,), jnp.int32),
            pltpu.SemaphoreType.DMA,
        ],
        compiler_params=pltpu.CompilerParams(needs_layout_passes=False),
    )


def kernel(vision_features, W, b, num_tokens):
    logits = jnp.squeeze(
        jnp.einsum('bsh,oh->bso', vision_features, W) + b, axis=-1)
    feat2d = vision_features.reshape(B * S, H)
    sel, idx, scores = _build_select()(feat2d, logits)
    idx = idx + (num_tokens - K)
    return (sel, idx, scores)


# speedup vs baseline: 1.2866x; 1.1398x over previous
import functools

import jax
import jax.numpy as jnp
import numpy as np
from jax import lax
from jax.experimental import pallas as pl
from jax.experimental.pallas import tpu as pltpu
from jax.experimental.pallas import tpu_sc as plsc

B = 32
S = 4096
H = 768
K = 256
NC = 2
NS = 16
SB = 2048
NV = S // 16
INT_MIN = np.int32(-2147483648)


def _score_body(w_ref, f_ref, o_ref):
    f = f_ref[0]
    w = w_ref[0]
    o_ref[...] = jnp.dot(f, w, preferred_element_type=jnp.float32)[None, None]


def _scores(features, W):
    return pl.pallas_call(
        _score_body,
        grid=(B, S // SB),
        in_specs=[
            pl.BlockSpec((1, H), lambda b, s: (0, 0)),
            pl.BlockSpec((1, SB, H), lambda b, s: (b, s, 0)),
        ],
        out_specs=pl.BlockSpec((1, 1, SB), lambda b, s: (b, 0, s)),
        out_shape=jax.ShapeDtypeStruct((B, 1, S), jnp.float32),
        compiler_params=pltpu.CompilerParams(
            dimension_semantics=("parallel", "arbitrary"),
        ),
    )(W, features).reshape(B, S)


def _sc_body(feat_hbm, logits_hbm, sel_hbm, idx_hbm, scr_hbm,
             logits_v, ukeys_v, aval_v, aidx_v, bval_v, bidx_v,
             gidx_v, rows_v, outsc_v, outix_v, sem, sem2):
    wid = lax.axis_index("s") * NC + lax.axis_index("c")
    iota16 = lax.iota(jnp.int32, 16)

    pltpu.sync_copy(logits_hbm.at[wid], logits_v)

    def p1(i, mvec):
        x = logits_v[pl.ds(i * 16, 16)]
        kb = lax.bitcast_convert_type(x, jnp.int32)
        u = kb ^ ((kb >> 31) | INT_MIN)
        ukeys_v[pl.ds(i * 16, 16)] = lax.bitcast_convert_type(u, jnp.uint32)
        return jnp.maximum(mvec, x)
    mvec = lax.fori_loop(0, NV, p1, jnp.full((16,), -jnp.inf, jnp.float32))
    m = jnp.max(mvec)

    def p2(i, dvec):
        x = logits_v[pl.ds(i * 16, 16)]
        return dvec + jnp.exp(x - m)
    dvec = lax.fori_loop(0, NV, p2, jnp.zeros((16,), jnp.float32))
    denom = jnp.sum(dvec)

    def bs(bi, t):
        cand = t | (jnp.uint32(1) << (jnp.uint32(31) - bi.astype(jnp.uint32)))
        def cnt_body(i, cs):
            base = i * 128
            new = []
            for u_ix in range(8):
                u = ukeys_v[pl.ds(base + u_ix * 16, 16)]
                new.append(cs[u_ix] + jnp.where(u >= cand, 1, 0).astype(jnp.int32))
            return tuple(new)
        zeros = tuple(jnp.zeros((16,), jnp.int32) for _ in range(8))
        cvecs = lax.fori_loop(0, NV // 8, cnt_body, zeros)
        cvec = cvecs[0]
        for u_ix in range(1, 8):
            cvec = cvec + cvecs[u_ix]
        cnt = jnp.sum(cvec)
        return jnp.where(cnt >= K, cand, t)
    t = lax.fori_loop(0, 32, bs, jnp.uint32(0))

    def comp_gt(i, off):
        u = ukeys_v[pl.ds(i * 16, 16)]
        mask = u > t
        ones = jnp.where(mask, 1, 0).astype(jnp.int32)
        pos = off + plsc.cumsum(ones) - 1
        plsc.store_scatter(aval_v, [pos], logits_v[pl.ds(i * 16, 16)], mask=mask)
        plsc.store_scatter(aidx_v, [pos], i * 16 + iota16, mask=mask)
        return off + jnp.sum(ones)
    cnt_gt = lax.fori_loop(0, NV, comp_gt, jnp.int32(0))

    def comp_eq(i, off):
        u = ukeys_v[pl.ds(i * 16, 16)]
        mask = u == t
        ones = jnp.where(mask, 1, 0).astype(jnp.int32)
        pos = off + plsc.cumsum(ones) - 1
        wmask = mask & (pos < K)
        plsc.store_scatter(aval_v, [pos], logits_v[pl.ds(i * 16, 16)], mask=wmask)
        plsc.store_scatter(aidx_v, [pos], i * 16 + iota16, mask=wmask)
        return off + jnp.sum(ones)
    lax.fori_loop(0, NV, comp_eq, cnt_gt)

    stages = []
    k = 2
    while k <= K:
        j = k // 2
        while j >= 1:
            stages.append((k, j))
            j //= 2
        k *= 2
    assert len(stages) % 2 == 0
    bufs = ((aval_v, aidx_v), (bval_v, bidx_v))
    for si, (k, j) in enumerate(stages):
        sval, sidx = bufs[si % 2]
        dval, didx = bufs[(si + 1) % 2]
        def stage(g, _, j=j, k=k, sval=sval, sidx=sidx, dval=dval, didx=didx):
            ivec = g * 16 + iota16
            pvec = ivec ^ j
            av = sval[pl.ds(g * 16, 16)]
            ai = sidx[pl.ds(g * 16, 16)]
            bv = plsc.load_gather(sval, [pvec])
            bi = plsc.load_gather(sidx, [pvec])
            a_first = (av > bv) | ((av == bv) & (ai < bi))
            want_first = ((ivec & k) == 0) == ((ivec & j) == 0)
            cond = want_first == a_first
            dval[pl.ds(g * 16, 16)] = jnp.where(cond, av, bv)
            didx[pl.ds(g * 16, 16)] = jnp.where(cond, ai, bi)
            return 0
        lax.fori_loop(0, K // 16, stage, 0)

    def outp(g, _):
        v = aval_v[pl.ds(g * 16, 16)]
        outsc_v[pl.ds(g * 16, 16)] = jnp.exp(v - m) / denom
        outix_v[pl.ds(g * 16, 16)] = aidx_v[pl.ds(g * 16, 16)]
        return 0
    lax.fori_loop(0, K // 16, outp, 0)
    pltpu.sync_copy(outsc_v, scr_hbm.at[wid])
    pltpu.sync_copy(outix_v, idx_hbm.at[wid])

    base = wid * S
    for c in range(4):
        for g2 in range(4):
            gidx_v[c, pl.ds(g2 * 16, 16)] = (
                base + aidx_v[pl.ds(c * 64 + g2 * 16, 16)])
    sems = (sem, sem2)
    pend = [None, None]
    pend[0] = pltpu.async_copy(feat_hbm.at[gidx_v.at[0]], rows_v.at[0], sems[0])
    for c in range(4):
        if c + 1 < 4:
            nb = (c + 1) % 2
            pend[nb] = pltpu.async_copy(
                feat_hbm.at[gidx_v.at[c + 1]], rows_v.at[nb], sems[nb])
        pend[c % 2].wait()
        pltpu.sync_copy(rows_v.at[c % 2], sel_hbm.at[wid, pl.ds(c * 64, 64)])


@functools.cache
def _build_select():
    return pl.kernel(
        _sc_body,
        out_type=(
            jax.ShapeDtypeStruct((B, K, H), jnp.float32),
            jax.ShapeDtypeStruct((B, K), jnp.int32),
            jax.ShapeDtypeStruct((B, K), jnp.float32),
        ),
        mesh=plsc.VectorSubcoreMesh(core_axis_name="c", subcore_axis_name="s",
                                    num_cores=NC, num_subcores=NS),
        scratch_types=[
            pltpu.VMEM((S,), jnp.float32),
            pltpu.VMEM((S,), jnp.uint32),
            pltpu.VMEM((K + 16,), jnp.float32),
            pltpu.VMEM((K + 16,), jnp.int32),
            pltpu.VMEM((K,), jnp.float32),
            pltpu.VMEM((K,), jnp.int32),
            pltpu.VMEM((4, 64), jnp.int32),
            pltpu.VMEM((2, 64, H), jnp.float32),
            pltpu.VMEM((K,), jnp.float32),
            pltpu.VMEM((K,), jnp.int32),
            pltpu.SemaphoreType.DMA,
            pltpu.SemaphoreType.DMA,
        ],
        compiler_params=pltpu.CompilerParams(needs_layout_passes=False),
    )


def kernel(vision_features, W, b, num_tokens):
    logits = jnp.squeeze(
        jnp.einsum('bsh,oh->bso', vision_features, W) + b, axis=-1)
    feat2d = vision_features.reshape(B * S, H)
    sel, idx, scores = _build_select()(feat2d, logits)
    idx = idx + (num_tokens - K)
    return (sel, idx, scores)
